# Initial kernel scaffold; baseline (speedup 1.0000x reference)
import jax
import jax.numpy as jnp
from jax.experimental import pallas as pl


def kernel(x, edge_index, W_pre, b_pre, W_post, b_post, W_lin, b_lin):
    raise NotImplementedError("write your pallas kernel here")


# speedup vs baseline: 1.7350x; 1.7350x over previous
import jax
import jax.numpy as jnp
from jax.experimental import pallas as pl
from jax.experimental.pallas import tpu as pltpu
from functools import partial

_NEG = -3.0e38
_POS = 3.0e38


def _pre_kernel(x_ref, w1_ref, w2_ref, bpre_ref, a_ref, b_ref):
    x = x_ref[...]
    a_ref[...] = jnp.dot(x, w1_ref[...], preferred_element_type=jnp.float32) + bpre_ref[...]
    b_ref[...] = jnp.dot(x, w2_ref[...], preferred_element_type=jnp.float32)


def _edge_kernel(src_ref, dst_ref, b_ref, sum_ref, ss_ref, mx_ref, mn_ref, deg_ref, *, n_edges):
    @pl.when(pl.program_id(0) == 0)
    def _():
        sum_ref[...] = jnp.zeros_like(sum_ref)
        ss_ref[...] = jnp.zeros_like(ss_ref)
        mx_ref[...] = jnp.full(mx_ref.shape, _NEG, jnp.float32)
        mn_ref[...] = jnp.full(mn_ref.shape, _POS, jnp.float32)
        deg_ref[...] = jnp.zeros_like(deg_ref)

    def body(j, _):
        s = src_ref[j]
        d = dst_ref[j]
        row = b_ref[pl.ds(s, 1), :]
        sum_ref[pl.ds(d, 1), :] += row
        ss_ref[pl.ds(d, 1), :] += row * row
        mx_ref[pl.ds(d, 1), :] = jnp.maximum(mx_ref[pl.ds(d, 1), :], row)
        mn_ref[pl.ds(d, 1), :] = jnp.minimum(mn_ref[pl.ds(d, 1), :], row)
        deg_ref[pl.ds(d, 1), :] += 1.0
        return 0

    jax.lax.fori_loop(0, n_edges, body, 0)


def _post_kernel(x_ref, a_ref, sum_ref, ss_ref, mx_ref, mn_ref, deg_ref,
                 wp0_ref, wp1_ref, wp2_ref, wp3_ref, wp4_ref, bpost_ref,
                 wlin_ref, blin_ref, out_ref, *, avg_deg, deg_halves):
    a = a_ref[...]
    sb = sum_ref[...]
    ssb = ss_ref[...]
    deg = deg_ref[:, 0:1] / deg_halves
    deg_c = jnp.maximum(deg, 1.0)
    inv = 1.0 / deg_c
    has = deg > 0.0

    mean = (deg * a + sb) * inv
    mean_sq = (deg * (a * a) + 2.0 * (a * sb) + ssb) * inv
    var = jnp.maximum(mean_sq - mean * mean, 0.0)
    std = jnp.sqrt(var + 1e-5)
    mx = jnp.where(has, a + mx_ref[...], 0.0)
    mn = jnp.where(has, a + mn_ref[...], 0.0)

    scale = deg_c * (1.0 / avg_deg)
    f32 = jnp.float32
    acc = jnp.dot(x_ref[...], wp0_ref[...], preferred_element_type=f32)
    acc += jnp.dot(mean * scale, wp1_ref[...], preferred_element_type=f32)
    acc += jnp.dot(mx * scale, wp2_ref[...], preferred_element_type=f32)
    acc += jnp.dot(mn * scale, wp3_ref[...], preferred_element_type=f32)
    acc += jnp.dot(std * scale, wp4_ref[...], preferred_element_type=f32)
    acc += bpost_ref[...]
    out = jnp.dot(acc, wlin_ref[...], preferred_element_type=f32) + blin_ref[...]
    out_ref[...] = jnp.maximum(out, 0.0)


def kernel(x, edge_index, W_pre, b_pre, W_post, b_post, W_lin, b_lin):
    n, d = x.shape
    e = edge_index.shape[1]
    src = edge_index[0]
    dst = edge_index[1]
    w1 = W_pre[:d]
    w2 = W_pre[d:]

    row_tile = 1000 if n % 1000 == 0 else n
    n_tiles = n // row_tile

    a_mat, b_mat = pl.pallas_call(
        _pre_kernel,
        grid=(n_tiles,),
        in_specs=[
            pl.BlockSpec((row_tile, d), lambda i: (i, 0)),
            pl.BlockSpec((d, d), lambda i: (0, 0)),
            pl.BlockSpec((d, d), lambda i: (0, 0)),
            pl.BlockSpec((1, d), lambda i: (0, 0)),
        ],
        out_specs=[
            pl.BlockSpec((row_tile, d), lambda i: (i, 0)),
            pl.BlockSpec((row_tile, d), lambda i: (i, 0)),
        ],
        out_shape=[
            jax.ShapeDtypeStruct((n, d), jnp.float32),
            jax.ShapeDtypeStruct((n, d), jnp.float32),
        ],
    )(x, w1, w2, b_pre.reshape(1, d))

    chunk = 32768
    n_chunks = -(-e // chunk)
    e_pad = chunk * n_chunks
    n_pad = n + 8
    src_p = jnp.concatenate([src, jnp.zeros((e_pad - e,), jnp.int32)])
    dst_p = jnp.concatenate([dst, jnp.full((e_pad - e,), n, jnp.int32)])
    sum_b, ss_b, mx_b, mn_b, deg8 = pl.pallas_call(
        partial(_edge_kernel, n_edges=chunk),
        grid=(n_chunks,),
        in_specs=[
            pl.BlockSpec((chunk,), lambda c: (c,), memory_space=pltpu.SMEM),
            pl.BlockSpec((chunk,), lambda c: (c,), memory_space=pltpu.SMEM),
            pl.BlockSpec((n, d), lambda c: (0, 0)),
        ],
        out_specs=[
            pl.BlockSpec((n_pad, d), lambda c: (0, 0)),
            pl.BlockSpec((n_pad, d), lambda c: (0, 0)),
            pl.BlockSpec((n_pad, d), lambda c: (0, 0)),
            pl.BlockSpec((n_pad, d), lambda c: (0, 0)),
            pl.BlockSpec((n_pad, 8), lambda c: (0, 0)),
        ],
        out_shape=[
            jax.ShapeDtypeStruct((n_pad, d), jnp.float32),
            jax.ShapeDtypeStruct((n_pad, d), jnp.float32),
            jax.ShapeDtypeStruct((n_pad, d), jnp.float32),
            jax.ShapeDtypeStruct((n_pad, d), jnp.float32),
            jax.ShapeDtypeStruct((n_pad, 8), jnp.float32),
        ],
    )(src_p, dst_p, b_mat)
    sum_b, ss_b, mx_b, mn_b, deg8 = (
        sum_b[:n], ss_b[:n], mx_b[:n], mn_b[:n], deg8[:n])

    out = pl.pallas_call(
        partial(_post_kernel, avg_deg=32.0, deg_halves=1.0),
        grid=(n_tiles,),
        in_specs=[
            pl.BlockSpec((row_tile, d), lambda i: (i, 0)),
            pl.BlockSpec((row_tile, d), lambda i: (i, 0)),
            pl.BlockSpec((row_tile, d), lambda i: (i, 0)),
            pl.BlockSpec((row_tile, d), lambda i: (i, 0)),
            pl.BlockSpec((row_tile, d), lambda i: (i, 0)),
            pl.BlockSpec((row_tile, d), lambda i: (i, 0)),
            pl.BlockSpec((row_tile, 8), lambda i: (i, 0)),
            pl.BlockSpec((d, d), lambda i: (0, 0)),
            pl.BlockSpec((d, d), lambda i: (0, 0)),
            pl.BlockSpec((d, d), lambda i: (0, 0)),
            pl.BlockSpec((d, d), lambda i: (0, 0)),
            pl.BlockSpec((d, d), lambda i: (0, 0)),
            pl.BlockSpec((1, d), lambda i: (0, 0)),
            pl.BlockSpec((d, d), lambda i: (0, 0)),
            pl.BlockSpec((1, d), lambda i: (0, 0)),
        ],
        out_specs=pl.BlockSpec((row_tile, d), lambda i: (i, 0)),
        out_shape=jax.ShapeDtypeStruct((n, d), jnp.float32),
    )(x, a_mat, sum_b, ss_b, mx_b, mn_b, deg8,
      W_post[0 * d:1 * d], W_post[1 * d:2 * d], W_post[2 * d:3 * d],
      W_post[3 * d:4 * d], W_post[4 * d:5 * d], b_post.reshape(1, d),
      W_lin, b_lin.reshape(1, d))
    return out
